# Initial kernel scaffold; baseline (speedup 1.0000x reference)
import jax
import jax.numpy as jnp
from jax.experimental import pallas as pl


def kernel(queries, stat_keys, token_keys, values, stat_valid_lens, W_q_stat, W_q_token, W_k_stat, W_k_token, W_v, W_o):
    raise NotImplementedError("write your pallas kernel here")


# speedup vs baseline: 41.0369x; 41.0369x over previous
import math

import jax
import jax.numpy as jnp
from jax.experimental import pallas as pl
from jax.experimental.pallas import tpu as pltpu

_B, _Q, _S, _T = 8, 16, 128, 64
_D = 256
_H = 8
_HD = _D // _H
_TSEL = 16
_KSTAT = 8
_NEG = -1000000.0
_HQ = _H * _Q


def _attn_kernel(svl_ref, q_ref, sk_ref, tk_ref, va_ref,
                 wqs_ref, wqt_ref, wks_ref, wkt_ref, wv_ref, wo_ref,
                 out_ref):
    b = pl.program_id(0)
    f32 = jnp.float32

    q = q_ref[0]
    qs = jnp.dot(q, wqs_ref[:], preferred_element_type=f32)
    qt = jnp.dot(q, wqt_ref[:], preferred_element_type=f32)
    ks = jnp.dot(sk_ref[0], wks_ref[:], preferred_element_type=f32)

    row_iota = jax.lax.broadcasted_iota(jnp.int32, (_HQ, _D), 0)
    col_iota = jax.lax.broadcasted_iota(jnp.int32, (_HQ, _D), 1)
    head_mask = (row_iota // _Q) == (col_iota // _HD)
    qs_blk = jnp.where(head_mask, jnp.concatenate([qs] * _H, axis=0), 0.0)
    qt_blk = jnp.where(head_mask, jnp.concatenate([qt] * _H, axis=0), 0.0)

    scale = 1.0 / math.sqrt(_HD)
    dn_t = (((1,), (1,)), ((), ()))
    statT = jax.lax.dot_general(ks, qs_blk, dn_t,
                                preferred_element_type=f32) * scale

    s_iota = jax.lax.broadcasted_iota(jnp.int32, (_S, _HQ), 0)
    vl = svl_ref[b]
    statT = jnp.where(s_iota < vl, statT, _NEG)

    work = statT
    sel = jnp.zeros((_S, _HQ), dtype=jnp.bool_)
    for _ in range(_KSTAT):
        m = jnp.max(work, axis=0, keepdims=True)
        cand = jnp.where(work == m, s_iota, _S)
        i0 = jnp.min(cand, axis=0, keepdims=True)
        pick = s_iota == i0
        sel = jnp.logical_or(sel, pick)
        work = jnp.where(pick, 3.0 * _NEG, work)
    kept = jnp.where(sel, statT, _NEG)
    mx = jnp.max(kept, axis=0, keepdims=True)
    e = jnp.exp(kept - mx)
    stat_wT = e / jnp.sum(e, axis=0, keepdims=True)

    st = _S * _TSEL
    kt = jnp.dot(tk_ref[:].reshape(st, _D), wkt_ref[:],
                 preferred_element_type=f32)
    v = jnp.dot(va_ref[:].reshape(st, _D), wv_ref[:],
                preferred_element_type=f32)

    tscT = jax.lax.dot_general(kt, qt_blk, dn_t,
                               preferred_element_type=f32) * scale
    t3 = tscT.reshape(_S, _TSEL, _HQ)
    tmx = jnp.max(t3, axis=1, keepdims=True)
    te = jnp.exp(t3 - tmx)
    tw3 = te / jnp.sum(te, axis=1, keepdims=True)
    cwT = (tw3 * stat_wT.reshape(_S, 1, _HQ)).reshape(st, _HQ)

    dn_0 = (((0,), (0,)), ((), ()))
    o_hq = jax.lax.dot_general(cwT, v, dn_0,
                               preferred_element_type=f32)

    final = jnp.concatenate(
        [o_hq[h * _Q:(h + 1) * _Q, h * _HD:(h + 1) * _HD] for h in range(_H)],
        axis=1)
    out_ref[0] = jnp.dot(final, wo_ref[:], preferred_element_type=f32)


def _build_call(interpret=False):
    t_blk_idx = _T // _TSEL - 1
    w_spec = pl.BlockSpec((_D, _D), lambda b, svl: (0, 0))
    grid_spec = pltpu.PrefetchScalarGridSpec(
        num_scalar_prefetch=1,
        grid=(_B,),
        in_specs=[
            pl.BlockSpec((1, _Q, _D), lambda b, svl: (b, 0, 0)),
            pl.BlockSpec((1, _S, _D), lambda b, svl: (b, 0, 0)),
            pl.BlockSpec((_S, _TSEL, _D), lambda b, svl: (b, t_blk_idx, 0)),
            pl.BlockSpec((_S, _TSEL, _D), lambda b, svl: (b, t_blk_idx, 0)),
            w_spec, w_spec, w_spec, w_spec, w_spec, w_spec,
        ],
        out_specs=pl.BlockSpec((1, _Q, _D), lambda b, svl: (b, 0, 0)),
    )
    return pl.pallas_call(
        _attn_kernel,
        grid_spec=grid_spec,
        out_shape=jax.ShapeDtypeStruct((_B, _Q, _D), jnp.float32),
        compiler_params=pltpu.CompilerParams(
            dimension_semantics=("arbitrary",)),
        interpret=interpret,
    )


def kernel(queries, stat_keys, token_keys, values, stat_valid_lens,
           W_q_stat, W_q_token, W_k_stat, W_k_token, W_v, W_o):
    call = _build_call()
    return call(stat_valid_lens.astype(jnp.int32), queries, stat_keys,
                token_keys, values, W_q_stat, W_q_token, W_k_stat, W_k_token,
                W_v, W_o)
